# baseline (speedup 1.0000x reference)
import functools

import jax
import jax.numpy as jnp
from jax import lax
from jax.experimental import pallas as pl
from jax.experimental.pallas import tpu as pltpu
from jax.experimental.pallas import tpu_sc as plsc

_NUM_CORES = 2
_NUM_SUBCORES = 16
_NUM_WORKERS = _NUM_CORES * _NUM_SUBCORES
_LANES = 16


def _tc_matvec_body(tbl_ref, w_ref, b_ref, o_ref):
    r = jnp.dot(w_ref[...], tbl_ref[...], preferred_element_type=jnp.float32)
    o_ref[...] = r.reshape(o_ref.shape) + b_ref[0, 0]


def _tc_matvec(table, W, b, seq):
    vocab, emb = table.shape
    blkv = 16384
    grid = (vocab + blkv - 1) // blkv
    tT = table.T
    w_row = (W.astype(jnp.float32) / seq)
    b_scaled = jnp.reshape(b.astype(jnp.float32), (1, 1)) / seq
    return pl.pallas_call(
        _tc_matvec_body,
        grid=(grid,),
        in_specs=[
            pl.BlockSpec((emb, blkv), lambda i: (0, i)),
            pl.BlockSpec((1, emb), lambda i: (0, 0)),
            pl.BlockSpec(memory_space=pltpu.SMEM),
        ],
        out_specs=pl.BlockSpec((blkv,), lambda i: (i,)),
        out_shape=jax.ShapeDtypeStruct((vocab,), jnp.float32),
    )(tT, w_row, b_scaled)


def _sc_gather_sum(x, t):
    seq, batch = x.shape
    bpw = batch // _NUM_WORKERS
    rows = 4
    nch = seq // rows
    nidx = rows * bpw
    nseg = nidx // 128

    mesh = plsc.VectorSubcoreMesh(
        core_axis_name="c", subcore_axis_name="s",
        num_cores=_NUM_CORES, num_subcores=_NUM_SUBCORES,
    )

    @functools.partial(
        pl.kernel,
        out_type=jax.ShapeDtypeStruct((batch,), jnp.float32),
        mesh=mesh,
        scratch_types=[
            pltpu.VMEM((2, rows, bpw), jnp.int32),
            pltpu.VMEM((2, rows * bpw), jnp.float32),
            pltpu.VMEM((bpw,), jnp.float32),
            pltpu.SemaphoreType.DMA,
            pltpu.SemaphoreType.DMA,
            pltpu.SemaphoreType.DMA,
            pltpu.SemaphoreType.DMA,
        ],
    )
    def sc_kernel(x_hbm, t_hbm, out_hbm, idx_v, vals_v, acc_v,
                  sx0, sx1, sg0, sg1):
        wid = lax.axis_index("s") * _NUM_CORES + lax.axis_index("c")
        base = wid * bpw
        sx = (sx0, sx1)
        sg = (sg0, sg1)
        nacc = bpw // _LANES

        def fire_x(i, buf):
            pltpu.async_copy(
                x_hbm.at[pl.ds(i * rows, rows), pl.ds(base, bpw)],
                idx_v.at[buf], sx[buf])

        def wait_x(buf):
            pltpu.make_async_copy(
                x_hbm.at[pl.ds(0, rows), pl.ds(base, bpw)],
                idx_v.at[buf], sx[buf]).wait()

        def fire_g(buf):
            for r in range(rows):
                for k in range(bpw // 128):
                    pltpu.async_copy(
                        t_hbm.at[idx_v.at[buf, r, pl.ds(k * 128, 128)]],
                        vals_v.at[buf, pl.ds(r * bpw + k * 128, 128)],
                        sg[buf])

        def wait_g(buf):
            pltpu.make_async_copy(
                t_hbm.at[pl.ds(0, nidx)], vals_v.at[buf], sg[buf]).wait()

        def accumulate(buf, acc):
            out = []
            for l in range(nacc):
                v = acc[l]
                for r in range(rows):
                    v = v + vals_v[buf, pl.ds(r * bpw + l * _LANES, _LANES)]
                out.append(v)
            return tuple(out)

        fire_x(0, 0)

        zero = jnp.zeros((_LANES,), jnp.float32)

        @pl.loop(0, nch // 2, init_carry=(zero,) * nacc)
        def _pair(j, acc):
            a = 2 * j
            wait_x(0)
            fire_g(0)
            fire_x(a + 1, 1)
            wait_g(0)
            wait_x(1)
            fire_g(1)

            @pl.when(a + 2 < nch)
            def _prefetch():
                fire_x(a + 2, 0)

            acc = accumulate(0, acc)
            wait_g(1)
            return accumulate(1, acc)

        acc = _pair
        if nch % 2 == 1:
            wait_x(0)
            fire_g(0)
            wait_g(0)
            acc = accumulate(0, acc)

        for l in range(nacc):
            acc_v[pl.ds(l * _LANES, _LANES)] = acc[l]
        pltpu.sync_copy(acc_v, out_hbm.at[pl.ds(base, bpw)])

    return sc_kernel(x, t)


def kernel(x, table, W, b):
    seq, _ = x.shape
    t = _tc_matvec(table, W, b, seq)
    return _sc_gather_sum(x, t)


# speedup vs baseline: 1.0248x; 1.0248x over previous
import functools

import jax
import jax.numpy as jnp
from jax import lax
from jax.experimental import pallas as pl
from jax.experimental.pallas import tpu as pltpu
from jax.experimental.pallas import tpu_sc as plsc

_NUM_CORES = 2
_NUM_SUBCORES = 16
_NUM_WORKERS = _NUM_CORES * _NUM_SUBCORES
_LANES = 16


def _tc_matvec_body(tbl_ref, w_ref, b_ref, o_ref):
    r = jnp.dot(w_ref[...], tbl_ref[...], preferred_element_type=jnp.float32)
    o_ref[...] = r.reshape(o_ref.shape) + b_ref[0, 0]


def _tc_matvec(table, W, b, seq):
    vocab, emb = table.shape
    blkv = 16384
    grid = (vocab + blkv - 1) // blkv
    tT = table.T
    w_row = (W.astype(jnp.float32) / seq)
    b_scaled = jnp.reshape(b.astype(jnp.float32), (1, 1)) / seq
    return pl.pallas_call(
        _tc_matvec_body,
        grid=(grid,),
        in_specs=[
            pl.BlockSpec((emb, blkv), lambda i: (0, i)),
            pl.BlockSpec((1, emb), lambda i: (0, 0)),
            pl.BlockSpec(memory_space=pltpu.SMEM),
        ],
        out_specs=pl.BlockSpec((blkv,), lambda i: (i,)),
        out_shape=jax.ShapeDtypeStruct((vocab,), jnp.float32),
    )(tT, w_row, b_scaled)


def _sc_gather_sum(x, t):
    seq, batch = x.shape
    bpw = batch // _NUM_WORKERS
    rows = 4
    nch = seq // rows
    nidx = rows * bpw
    nseg = nidx // 128

    mesh = plsc.VectorSubcoreMesh(
        core_axis_name="c", subcore_axis_name="s",
        num_cores=_NUM_CORES, num_subcores=_NUM_SUBCORES,
    )

    @functools.partial(
        pl.kernel,
        out_type=jax.ShapeDtypeStruct((batch,), jnp.float32),
        mesh=mesh,
        scratch_types=[
            pltpu.VMEM((2, rows, bpw), jnp.int32),
            pltpu.VMEM((2, rows * bpw), jnp.float32),
            pltpu.VMEM((bpw,), jnp.float32),
            pltpu.SemaphoreType.DMA,
            pltpu.SemaphoreType.DMA,
            pltpu.SemaphoreType.DMA,
            pltpu.SemaphoreType.DMA,
        ],
    )
    def sc_kernel(x_hbm, t_hbm, out_hbm, idx_v, vals_v, acc_v,
                  sx0, sx1, sg0, sg1):
        wid = lax.axis_index("s") * _NUM_CORES + lax.axis_index("c")
        base = wid * bpw
        sx = (sx0, sx1)
        sg = (sg0, sg1)
        nacc = bpw // _LANES

        def fire_x(i, buf):
            pltpu.async_copy(
                x_hbm.at[pl.ds(i * rows, rows), pl.ds(base, bpw)],
                idx_v.at[buf], sx[buf])

        def wait_x(buf):
            pltpu.make_async_copy(
                x_hbm.at[pl.ds(0, rows), pl.ds(base, bpw)],
                idx_v.at[buf], sx[buf]).wait()

        def fire_g(buf):
            for r in range(rows):
                for k in range(bpw // 128):
                    pltpu.async_copy(
                        t_hbm.at[idx_v.at[buf, r, pl.ds(k * 128, 128)]],
                        vals_v.at[buf, pl.ds(r * bpw + k * 128, 128)],
                        sg[buf])

        def wait_g(buf):
            pltpu.make_async_copy(
                t_hbm.at[pl.ds(0, nidx)], vals_v.at[buf], sg[buf]).wait()

        def accumulate(buf):
            for l in range(nacc):
                v = acc_v[pl.ds(l * _LANES, _LANES)]
                for r in range(rows):
                    v = v + vals_v[buf, pl.ds(r * bpw + l * _LANES, _LANES)]
                acc_v[pl.ds(l * _LANES, _LANES)] = v

        zero = jnp.zeros((_LANES,), jnp.float32)
        for l in range(nacc):
            acc_v[pl.ds(l * _LANES, _LANES)] = zero

        fire_x(0, 0)

        @pl.loop(0, nch // 2)
        def _pair(j):
            a = 2 * j
            wait_x(0)
            fire_g(0)
            fire_x(a + 1, 1)
            wait_g(0)
            wait_x(1)
            fire_g(1)

            @pl.when(a + 2 < nch)
            def _prefetch():
                fire_x(a + 2, 0)

            accumulate(0)
            wait_g(1)
            accumulate(1)

        if nch % 2 == 1:
            wait_x(0)
            fire_g(0)
            wait_g(0)
            accumulate(0)

        pltpu.sync_copy(acc_v, out_hbm.at[pl.ds(base, bpw)])

    return sc_kernel(x, t)


def kernel(x, table, W, b):
    seq, _ = x.shape
    t = _tc_matvec(table, W, b, seq)
    return _sc_gather_sum(x, t)


# speedup vs baseline: 1.1493x; 1.1215x over previous
import functools

import jax
import jax.numpy as jnp
from jax import lax
from jax.experimental import pallas as pl
from jax.experimental.pallas import tpu as pltpu
from jax.experimental.pallas import tpu_sc as plsc

_NUM_CORES = 2
_NUM_SUBCORES = 16
_NUM_WORKERS = _NUM_CORES * _NUM_SUBCORES
_LANES = 16


def _tc_matvec_body(tbl_ref, w_ref, b_ref, o_ref):
    r = jnp.dot(w_ref[...], tbl_ref[...], preferred_element_type=jnp.float32)
    o_ref[...] = r.reshape(o_ref.shape) + b_ref[0, 0]


def _tc_matvec(table, W, b, seq):
    vocab, emb = table.shape
    blkv = 16384
    grid = (vocab + blkv - 1) // blkv
    tT = table.T
    w_row = (W.astype(jnp.float32) / seq)
    b_scaled = jnp.reshape(b.astype(jnp.float32), (1, 1)) / seq
    return pl.pallas_call(
        _tc_matvec_body,
        grid=(grid,),
        in_specs=[
            pl.BlockSpec((emb, blkv), lambda i: (0, i)),
            pl.BlockSpec((1, emb), lambda i: (0, 0)),
            pl.BlockSpec(memory_space=pltpu.SMEM),
        ],
        out_specs=pl.BlockSpec((blkv,), lambda i: (i,)),
        out_shape=jax.ShapeDtypeStruct((vocab,), jnp.float32),
    )(tT, w_row, b_scaled)


def _sc_gather_sum(x, t):
    seq, batch = x.shape
    bpw = batch // _NUM_WORKERS
    rows = 4
    nch = seq // rows
    nidx = rows * bpw
    nseg = nidx // 128

    mesh = plsc.VectorSubcoreMesh(
        core_axis_name="c", subcore_axis_name="s",
        num_cores=_NUM_CORES, num_subcores=_NUM_SUBCORES,
    )

    @functools.partial(
        pl.kernel,
        out_type=jax.ShapeDtypeStruct((batch,), jnp.float32),
        mesh=mesh,
        scratch_types=[
            pltpu.VMEM((2, rows, bpw), jnp.int32),
            pltpu.VMEM((2, rows * bpw), jnp.float32),
            pltpu.VMEM((bpw,), jnp.float32),
            pltpu.SemaphoreType.DMA,
            pltpu.SemaphoreType.DMA,
            pltpu.SemaphoreType.DMA,
            pltpu.SemaphoreType.DMA,
        ],
    )
    def sc_kernel(x_hbm, t_hbm, out_hbm, idx_v, vals_v, acc_v,
                  sx0, sx1, sg0, sg1):
        wid = lax.axis_index("s") * _NUM_CORES + lax.axis_index("c")
        base = wid * bpw
        sx = (sx0, sx1)
        sg = (sg0, sg1)
        nacc = bpw // _LANES

        def fire_x(i, buf):
            pltpu.async_copy(
                x_hbm.at[pl.ds(i * rows, rows), pl.ds(base, bpw)],
                idx_v.at[buf], sx[buf])

        def wait_x(buf):
            pltpu.make_async_copy(
                x_hbm.at[pl.ds(0, rows), pl.ds(base, bpw)],
                idx_v.at[buf], sx[buf]).wait()

        def fire_g(buf):
            for r in range(rows):
                for k in range(bpw // 128):
                    pltpu.async_copy(
                        t_hbm.at[idx_v.at[buf, r, pl.ds(k * 128, 128)]],
                        vals_v.at[buf, pl.ds(r * bpw + k * 128, 128)],
                        sg[buf])

        def wait_g(buf):
            pltpu.make_async_copy(
                t_hbm.at[pl.ds(0, nidx)], vals_v.at[buf], sg[buf]).wait()

        def accumulate(buf):
            for l in range(nacc):
                v = acc_v[pl.ds(l * _LANES, _LANES)]
                for r in range(rows):
                    v = v + vals_v[buf, pl.ds(r * bpw + l * _LANES, _LANES)]
                acc_v[pl.ds(l * _LANES, _LANES)] = v

        zero = jnp.zeros((_LANES,), jnp.float32)
        for l in range(nacc):
            acc_v[pl.ds(l * _LANES, _LANES)] = zero

        fire_x(0, 0)
        wait_x(0)
        fire_g(0)
        if nch > 1:
            fire_x(1, 1)

        @pl.loop(0, nch // 2)
        def _pair(j):
            a = 2 * j
            wait_x(1)
            fire_g(1)
            wait_g(0)

            @pl.when(a + 2 < nch)
            def _pf_x0():
                fire_x(a + 2, 0)

            accumulate(0)

            @pl.when(a + 2 < nch)
            def _pf_g0():
                wait_x(0)
                fire_g(0)

            wait_g(1)

            @pl.when(a + 3 < nch)
            def _pf_x1():
                fire_x(a + 3, 1)

            accumulate(1)

        if nch % 2 == 1:
            wait_g(0)
            accumulate(0)

        pltpu.sync_copy(acc_v, out_hbm.at[pl.ds(base, bpw)])

    return sc_kernel(x, t)


def kernel(x, table, W, b):
    seq, _ = x.shape
    t = _tc_matvec(table, W, b, seq)
    return _sc_gather_sum(x, t)


# speedup vs baseline: 1.2026x; 1.0463x over previous
import functools

import jax
import jax.numpy as jnp
from jax import lax
from jax.experimental import pallas as pl
from jax.experimental.pallas import tpu as pltpu
from jax.experimental.pallas import tpu_sc as plsc

_NUM_CORES = 2
_NUM_SUBCORES = 16
_NUM_WORKERS = _NUM_CORES * _NUM_SUBCORES
_LANES = 16


def _tc_matvec_body(tbl_ref, w_ref, b_ref, o_ref):
    r = jnp.dot(w_ref[...], tbl_ref[...], preferred_element_type=jnp.float32)
    o_ref[...] = r.reshape(o_ref.shape) + b_ref[0, 0]


def _tc_matvec(table, W, b, seq):
    vocab, emb = table.shape
    blkv = 32768
    grid = (vocab + blkv - 1) // blkv
    tT = table.T
    w_row = (W.astype(jnp.float32) / seq)
    b_scaled = jnp.reshape(b.astype(jnp.float32), (1, 1)) / seq
    return pl.pallas_call(
        _tc_matvec_body,
        grid=(grid,),
        in_specs=[
            pl.BlockSpec((emb, blkv), lambda i: (0, i)),
            pl.BlockSpec((1, emb), lambda i: (0, 0)),
            pl.BlockSpec(memory_space=pltpu.SMEM),
        ],
        out_specs=pl.BlockSpec((blkv,), lambda i: (i,)),
        out_shape=jax.ShapeDtypeStruct((vocab,), jnp.float32),
    )(tT, w_row, b_scaled)


def _sc_gather_sum(x, t):
    seq, batch = x.shape
    bpw = batch // _NUM_WORKERS
    rows = 4
    nch = seq // rows
    nidx = rows * bpw
    nseg = nidx // 128

    mesh = plsc.VectorSubcoreMesh(
        core_axis_name="c", subcore_axis_name="s",
        num_cores=_NUM_CORES, num_subcores=_NUM_SUBCORES,
    )

    @functools.partial(
        pl.kernel,
        out_type=jax.ShapeDtypeStruct((batch,), jnp.float32),
        mesh=mesh,
        scratch_types=[
            pltpu.VMEM((2, rows, bpw), jnp.int32),
            pltpu.VMEM((2, rows * bpw), jnp.float32),
            pltpu.VMEM((bpw,), jnp.float32),
            pltpu.SemaphoreType.DMA,
            pltpu.SemaphoreType.DMA,
            pltpu.SemaphoreType.DMA,
            pltpu.SemaphoreType.DMA,
        ],
    )
    def sc_kernel(x_hbm, t_hbm, out_hbm, idx_v, vals_v, acc_v,
                  sx0, sx1, sg0, sg1):
        wid = lax.axis_index("s") * _NUM_CORES + lax.axis_index("c")
        base = wid * bpw
        sx = (sx0, sx1)
        sg = (sg0, sg1)
        nacc = bpw // _LANES

        def fire_x(i, buf):
            pltpu.async_copy(
                x_hbm.at[pl.ds(i * rows, rows), pl.ds(base, bpw)],
                idx_v.at[buf], sx[buf])

        def wait_x(buf):
            pltpu.make_async_copy(
                x_hbm.at[pl.ds(0, rows), pl.ds(base, bpw)],
                idx_v.at[buf], sx[buf]).wait()

        def fire_g(buf):
            for r in range(rows):
                for k in range(bpw // 128):
                    pltpu.async_copy(
                        t_hbm.at[idx_v.at[buf, r, pl.ds(k * 128, 128)]],
                        vals_v.at[buf, pl.ds(r * bpw + k * 128, 128)],
                        sg[buf])

        def wait_g(buf):
            pltpu.make_async_copy(
                t_hbm.at[pl.ds(0, nidx)], vals_v.at[buf], sg[buf]).wait()

        def accumulate(buf):
            for l in range(nacc):
                v = acc_v[pl.ds(l * _LANES, _LANES)]
                for r in range(rows):
                    v = v + vals_v[buf, pl.ds(r * bpw + l * _LANES, _LANES)]
                acc_v[pl.ds(l * _LANES, _LANES)] = v

        zero = jnp.zeros((_LANES,), jnp.float32)
        for l in range(nacc):
            acc_v[pl.ds(l * _LANES, _LANES)] = zero

        fire_x(0, 0)
        wait_x(0)
        fire_g(0)
        if nch > 1:
            fire_x(1, 1)

        @pl.loop(0, nch // 2)
        def _pair(j):
            a = 2 * j
            wait_x(1)
            fire_g(1)
            wait_g(0)

            @pl.when(a + 2 < nch)
            def _pf_x0():
                fire_x(a + 2, 0)

            accumulate(0)

            @pl.when(a + 2 < nch)
            def _pf_g0():
                wait_x(0)
                fire_g(0)

            wait_g(1)

            @pl.when(a + 3 < nch)
            def _pf_x1():
                fire_x(a + 3, 1)

            accumulate(1)

        if nch % 2 == 1:
            wait_g(0)
            accumulate(0)

        pltpu.sync_copy(acc_v, out_hbm.at[pl.ds(base, bpw)])

    return sc_kernel(x, t)


def kernel(x, table, W, b):
    seq, _ = x.shape
    t = _tc_matvec(table, W, b, seq)
    return _sc_gather_sum(x, t)
